# baseline (speedup 1.0000x reference)
import jax
import jax.numpy as jnp
from jax import lax
from jax.experimental import pallas as pl
from jax.experimental.pallas import tpu as pltpu

_HEADS = 12
_DH = 64
_N = 4096
_H2 = _N // 2
_D = 768
_LOGN = 12
_TILE = 512
_NT2 = _H2 // _TILE
_NT = _N // _TILE


def _mm_t(a, b):
    return lax.dot_general(a, b, (((1,), (1,)), ((), ())),
                           preferred_element_type=jnp.float32)


def _mm(a, b):
    return lax.dot_general(a, b, (((1,), (0,)), ((), ())),
                           preferred_element_type=jnp.float32)


def _v_kernel(x_ref, wv_ref, o_ref):
    o_ref[...] = _mm_t(x_ref[...].astype(jnp.bfloat16),
                       wv_ref[...]).astype(jnp.bfloat16)


def _head_mats():
    scale = jnp.float32(_DH ** -0.5)
    seg = (lax.broadcasted_iota(jnp.int32, (_D, _HEADS), 0) // _DH ==
           lax.broadcasted_iota(jnp.int32, (_D, _HEADS), 1))
    smat = jnp.where(seg, scale, jnp.float32(0.0)).astype(jnp.bfloat16)
    rep = (lax.broadcasted_iota(jnp.int32, (_HEADS, _D), 0) ==
           lax.broadcasted_iota(jnp.int32, (_HEADS, _D), 1) // _DH)
    bmat = jnp.where(rep, jnp.float32(1.0),
                     jnp.float32(0.0)).astype(jnp.bfloat16)
    return smat, bmat


def _swap_sub(x, st):
    n = x.shape[0]
    down = pltpu.roll(x, st, 0)
    up = pltpu.roll(x, n - st, 0)
    rows = lax.broadcasted_iota(jnp.int32, (n, 1), 0)
    bit = (rows & st) != 0
    return jnp.where(bit, down, up)


def _stage_kernel(v_ref, wqk_ref, out_ref, ha_ref, hb_ref, vs_ref,
                  ra_ref, rb_ref):
    s = pl.program_id(0)

    @pl.when(s == 0)
    def _():
        def cbody(t, carry):
            rows = pl.ds(t * _TILE, _TILE)
            out_ref[rows, :] = v_ref[rows, :]
            return carry

        lax.fori_loop(0, _NT, cbody, 0)

    wq = wqk_ref[0, :_D, :]
    wk = wqk_ref[0, _D:, :]
    smat, bmat = _head_mats()

    @pl.when(s < 4)
    def _():
        st = jnp.int32(1) << s

        def body(t, carry):
            rows = pl.ds(t * _TILE, _TILE)
            hb16 = out_ref[rows, :]
            h_t = hb16.astype(jnp.float32)
            hd = pltpu.roll(h_t, st, 0)
            ridx = lax.broadcasted_iota(jnp.int32, (_TILE, 1), 0)
            bit = (ridx & st) != 0
            ha_t = jnp.where(bit, hd, h_t).astype(jnp.bfloat16)
            qa = _mm_t(ha_t, wq)
            k = _mm_t(hb16, wk)
            e_t = _mm((qa * k).astype(jnp.bfloat16), smat)
            es_t = _swap_sub(e_t, st)
            w_t = jax.nn.sigmoid(e_t - es_t)
            wf = _mm(w_t.astype(jnp.bfloat16), bmat)
            v_t = v_ref[rows, :].astype(jnp.float32)
            vs_t = v_t + _swap_sub(v_t, st)
            out_ref[rows, :] = (wf * vs_t).astype(jnp.bfloat16)
            return carry

        lax.fori_loop(0, _NT, body, 0)

    @pl.when(s >= 4)
    def _():
        for c in range(4, _LOGN):
            @pl.when(s == c)
            def _(c=c):
                st = 1 << c
                g = _N // (2 * st)

                def gbody(i, carry):
                    dst = pl.ds(i * st, st)
                    a0 = i * 2 * st
                    ha_ref[dst, :] = out_ref[pl.ds(a0, st), :]
                    hb_ref[dst, :] = out_ref[pl.ds(a0 + st, st), :]
                    vs_ref[dst, :] = (v_ref[pl.ds(a0, st), :] +
                                      v_ref[pl.ds(a0 + st, st), :])
                    return carry

                lax.fori_loop(0, g, gbody, 0)

        def mid(t, carry):
            rows = pl.ds(t * _TILE, _TILE)
            ha_t = ha_ref[rows, :]
            hb_t = hb_ref[rows, :]
            qa = _mm_t(ha_t, wq)
            ka = _mm_t(ha_t, wk)
            kb = _mm_t(hb_t, wk)
            e0 = _mm((qa * ka).astype(jnp.bfloat16), smat)
            e1 = _mm((qa * kb).astype(jnp.bfloat16), smat)
            w0 = jax.nn.sigmoid(e0 - e1)
            w1 = 1.0 - w0
            wf0 = _mm(w0.astype(jnp.bfloat16), bmat)
            wf1 = _mm(w1.astype(jnp.bfloat16), bmat)
            vs_t = vs_ref[rows, :].astype(jnp.float32)
            ra_ref[rows, :] = (wf0 * vs_t).astype(jnp.bfloat16)
            rb_ref[rows, :] = (wf1 * vs_t).astype(jnp.bfloat16)
            return carry

        lax.fori_loop(0, _NT2, mid, 0)

        for c in range(4, _LOGN):
            @pl.when(s == c)
            def _(c=c):
                st = 1 << c
                g = _N // (2 * st)

                def sbody(i, carry):
                    src = pl.ds(i * st, st)
                    a0 = i * 2 * st
                    out_ref[pl.ds(a0, st), :] = ra_ref[src, :]
                    out_ref[pl.ds(a0 + st, st), :] = rb_ref[src, :]
                    return carry

                lax.fori_loop(0, g, sbody, 0)


def _run(x2, W_v, W_qk, interpret=False):
    v = pl.pallas_call(
        _v_kernel,
        grid=(_NT,),
        in_specs=[pl.BlockSpec((_TILE, _D), lambda i: (i, 0)),
                  pl.BlockSpec((_D, _D), lambda i: (0, 0))],
        out_specs=pl.BlockSpec((_TILE, _D), lambda i: (i, 0)),
        out_shape=jax.ShapeDtypeStruct((_N, _D), jnp.bfloat16),
        interpret=interpret,
    )(x2, W_v.astype(jnp.bfloat16))

    h = pl.pallas_call(
        _stage_kernel,
        grid=(_LOGN,),
        in_specs=[pl.BlockSpec((_N, _D), lambda s: (0, 0)),
                  pl.BlockSpec((1, 2 * _D, _D), lambda s: (s, 0, 0))],
        out_specs=pl.BlockSpec((_N, _D), lambda s: (0, 0)),
        out_shape=jax.ShapeDtypeStruct((_N, _D), jnp.bfloat16),
        scratch_shapes=[pltpu.VMEM((_H2, _D), jnp.bfloat16),
                        pltpu.VMEM((_H2, _D), jnp.bfloat16),
                        pltpu.VMEM((_H2, _D), jnp.bfloat16),
                        pltpu.VMEM((_H2, _D), jnp.bfloat16),
                        pltpu.VMEM((_H2, _D), jnp.bfloat16)],
        interpret=interpret,
    )(v, W_qk.astype(jnp.bfloat16))
    return h


def kernel(x, W_v, W_qk):
    B, N, D = x.shape
    h = _run(x.reshape(N, D), W_v, W_qk)
    return h.astype(jnp.float32).reshape(B, N, D)


# speedup vs baseline: 1.4109x; 1.4109x over previous
import jax
import jax.numpy as jnp
from jax import lax
from jax.experimental import pallas as pl
from jax.experimental.pallas import tpu as pltpu

_HEADS = 12
_DH = 64
_N = 4096
_H2 = _N // 2
_D = 768
_LOGN = 12
_TILE = 512
_NT2 = _H2 // _TILE
_NT = _N // _TILE


def _mm_t(a, b):
    return lax.dot_general(a, b, (((1,), (1,)), ((), ())),
                           preferred_element_type=jnp.float32)


def _mm(a, b):
    return lax.dot_general(a, b, (((1,), (0,)), ((), ())),
                           preferred_element_type=jnp.float32)


def _v_kernel(x_ref, wv_ref, o_ref):
    o_ref[...] = _mm_t(x_ref[...].astype(jnp.bfloat16),
                       wv_ref[...]).astype(jnp.bfloat16)


def _head_mats():
    scale = jnp.float32(_DH ** -0.5)
    seg = (lax.broadcasted_iota(jnp.int32, (_D, _HEADS), 0) // _DH ==
           lax.broadcasted_iota(jnp.int32, (_D, _HEADS), 1))
    smat = jnp.where(seg, scale, jnp.float32(0.0)).astype(jnp.bfloat16)
    rep = (lax.broadcasted_iota(jnp.int32, (_HEADS, _D), 0) ==
           lax.broadcasted_iota(jnp.int32, (_HEADS, _D), 1) // _DH)
    bmat = jnp.where(rep, jnp.float32(1.0),
                     jnp.float32(0.0)).astype(jnp.bfloat16)
    return smat, bmat


def _swap_sub(x, st):
    n = x.shape[0]
    down = pltpu.roll(x, st, 0)
    up = pltpu.roll(x, n - st, 0)
    rows = lax.broadcasted_iota(jnp.int32, (n, 1), 0)
    bit = (rows & st) != 0
    return jnp.where(bit, down, up)


def _stage_kernel(v_ref, wqk_ref, out_ref, ha_ref, hb_ref, vs_ref,
                  ra_ref, rb_ref):
    s = pl.program_id(0)

    @pl.when(s == 0)
    def _():
        def cbody(t, carry):
            rows = pl.ds(t * _TILE, _TILE)
            out_ref[rows, :] = v_ref[rows, :]
            return carry

        lax.fori_loop(0, _NT, cbody, 0)

    wq = wqk_ref[0, :_D, :]
    wk = wqk_ref[0, _D:, :]
    smat, bmat = _head_mats()

    for c in range(4):
        @pl.when(s == c)
        def _(c=c):
            st = 1 << c

            def body(t, carry):
                rows = pl.ds(t * _TILE, _TILE)
                hb16 = out_ref[rows, :]
                h_t = hb16.astype(jnp.float32)
                hd = pltpu.roll(h_t, st, 0)
                ridx = lax.broadcasted_iota(jnp.int32, (_TILE, 1), 0)
                bit = (ridx & st) != 0
                ha_t = jnp.where(bit, hd, h_t).astype(jnp.bfloat16)
                qa = _mm_t(ha_t, wq)
                k = _mm_t(hb16, wk)
                e_t = _mm((qa * k).astype(jnp.bfloat16), smat)
                es_t = _swap_sub(e_t, st)
                w_t = jax.nn.sigmoid(e_t - es_t)
                wf = _mm(w_t.astype(jnp.bfloat16), bmat)
                v_t = v_ref[rows, :].astype(jnp.float32)
                vs_t = v_t + _swap_sub(v_t, st)
                out_ref[rows, :] = (wf * vs_t).astype(jnp.bfloat16)
                return carry

            lax.fori_loop(0, _NT, body, 0)

    @pl.when(s >= 4)
    def _():
        for c in range(4, _LOGN):
            @pl.when(s == c)
            def _(c=c):
                st = 1 << c
                g = _N // (2 * st)

                def gbody(i, carry):
                    dst = pl.ds(i * st, st)
                    a0 = i * 2 * st
                    ha_ref[dst, :] = out_ref[pl.ds(a0, st), :]
                    hb_ref[dst, :] = out_ref[pl.ds(a0 + st, st), :]
                    vs_ref[dst, :] = (v_ref[pl.ds(a0, st), :] +
                                      v_ref[pl.ds(a0 + st, st), :])
                    return carry

                lax.fori_loop(0, g, gbody, 0)

        def mid(t, carry):
            rows = pl.ds(t * _TILE, _TILE)
            ha_t = ha_ref[rows, :]
            hb_t = hb_ref[rows, :]
            qa = _mm_t(ha_t, wq)
            ka = _mm_t(ha_t, wk)
            kb = _mm_t(hb_t, wk)
            e0 = _mm((qa * ka).astype(jnp.bfloat16), smat)
            e1 = _mm((qa * kb).astype(jnp.bfloat16), smat)
            w0 = jax.nn.sigmoid(e0 - e1)
            w1 = 1.0 - w0
            wf0 = _mm(w0.astype(jnp.bfloat16), bmat)
            wf1 = _mm(w1.astype(jnp.bfloat16), bmat)
            vs_t = vs_ref[rows, :].astype(jnp.float32)
            ra_ref[rows, :] = (wf0 * vs_t).astype(jnp.bfloat16)
            rb_ref[rows, :] = (wf1 * vs_t).astype(jnp.bfloat16)
            return carry

        lax.fori_loop(0, _NT2, mid, 0)

        for c in range(4, _LOGN):
            @pl.when(s == c)
            def _(c=c):
                st = 1 << c
                g = _N // (2 * st)

                def sbody(i, carry):
                    src = pl.ds(i * st, st)
                    a0 = i * 2 * st
                    out_ref[pl.ds(a0, st), :] = ra_ref[src, :]
                    out_ref[pl.ds(a0 + st, st), :] = rb_ref[src, :]
                    return carry

                lax.fori_loop(0, g, sbody, 0)


def _run(x2, W_v, W_qk, interpret=False):
    v = pl.pallas_call(
        _v_kernel,
        grid=(_NT,),
        in_specs=[pl.BlockSpec((_TILE, _D), lambda i: (i, 0)),
                  pl.BlockSpec((_D, _D), lambda i: (0, 0))],
        out_specs=pl.BlockSpec((_TILE, _D), lambda i: (i, 0)),
        out_shape=jax.ShapeDtypeStruct((_N, _D), jnp.bfloat16),
        interpret=interpret,
    )(x2, W_v.astype(jnp.bfloat16))

    h = pl.pallas_call(
        _stage_kernel,
        grid=(_LOGN,),
        in_specs=[pl.BlockSpec((_N, _D), lambda s: (0, 0)),
                  pl.BlockSpec((1, 2 * _D, _D), lambda s: (s, 0, 0))],
        out_specs=pl.BlockSpec((_N, _D), lambda s: (0, 0)),
        out_shape=jax.ShapeDtypeStruct((_N, _D), jnp.bfloat16),
        scratch_shapes=[pltpu.VMEM((_H2, _D), jnp.bfloat16),
                        pltpu.VMEM((_H2, _D), jnp.bfloat16),
                        pltpu.VMEM((_H2, _D), jnp.bfloat16),
                        pltpu.VMEM((_H2, _D), jnp.bfloat16),
                        pltpu.VMEM((_H2, _D), jnp.bfloat16)],
        interpret=interpret,
    )(v, W_qk.astype(jnp.bfloat16))
    return h


def kernel(x, W_v, W_qk):
    B, N, D = x.shape
    h = _run(x.reshape(N, D), W_v, W_qk)
    return h.astype(jnp.float32).reshape(B, N, D)
